# baseline (device time: 26512 ns/iter reference)
import jax
import jax.numpy as jnp
from jax import lax
from jax.experimental import pallas as pl
from jax.experimental.pallas import tpu as pltpu

N_DEV = 4
N_LAYERS = 3


def kernel(x, Win0, Wout0, Win1, Wout1, Win2, Wout2):
    m, d = x.shape
    _, f = Win0.shape

    def body(x_ref, win0_ref, wout0_ref, win1_ref, wout1_ref, win2_ref,
             wout2_ref, out_ref, win_vmem, wout_vmem, comm_ref,
             load_sems, send_sems, recv_sems):
        my_pos = lax.axis_index("i")

        win_hbm = [win0_ref, win1_ref, win2_ref]
        wout_hbm = [wout0_ref, wout1_ref, wout2_ref]
        loads = []
        for k in range(N_LAYERS):
            ci = pltpu.make_async_copy(
                win_hbm[k], win_vmem.at[k], load_sems.at[2 * k])
            ci.start()
            co = pltpu.make_async_copy(
                wout_hbm[k], wout_vmem.at[k], load_sems.at[2 * k + 1])
            co.start()
            loads.append((ci, co))

        barrier_sem = pltpu.get_barrier_semaphore()
        for off in range(1, N_DEV):
            peer = (my_pos + off) % N_DEV
            pl.semaphore_signal(
                barrier_sem, inc=1,
                device_id=(peer,), device_id_type=pl.DeviceIdType.MESH,
            )
        pl.semaphore_wait(barrier_sem, N_DEV - 1)

        xb = x_ref[...].astype(jnp.bfloat16)
        loads[0][0].wait()
        wi = win_vmem[0].astype(jnp.bfloat16)
        loads[0][1].wait()
        wo = wout_vmem[0].astype(jnp.bfloat16)

        for k in range(N_LAYERS):
            h = jnp.dot(xb, wi, preferred_element_type=jnp.float32)
            h = jnp.maximum(h, 0.0).astype(jnp.bfloat16)
            p = jnp.dot(h, wo, preferred_element_type=jnp.float32)

            comm_ref[k, my_pos] = p.astype(jnp.bfloat16)
            sends = []
            for off in range(1, N_DEV):
                peer = (my_pos + off) % N_DEV
                rdma = pltpu.make_async_remote_copy(
                    src_ref=comm_ref.at[k, my_pos],
                    dst_ref=comm_ref.at[k, my_pos],
                    send_sem=send_sems.at[k, off - 1],
                    recv_sem=recv_sems.at[k, my_pos],
                    device_id=(peer,),
                    device_id_type=pl.DeviceIdType.MESH,
                )
                rdma.start()
                sends.append(rdma)

            if k + 1 < N_LAYERS:
                loads[k + 1][0].wait()
                wi = win_vmem[k + 1].astype(jnp.bfloat16)
                loads[k + 1][1].wait()
                wo = wout_vmem[k + 1].astype(jnp.bfloat16)

            for off in range(1, N_DEV):
                sender = (my_pos + off) % N_DEV
                recv = pltpu.make_async_remote_copy(
                    src_ref=comm_ref.at[k, sender],
                    dst_ref=comm_ref.at[k, sender],
                    send_sem=send_sems.at[k, off - 1],
                    recv_sem=recv_sems.at[k, sender],
                    device_id=(my_pos,),
                    device_id_type=pl.DeviceIdType.MESH,
                )
                recv.wait_recv()
            for rdma in sends:
                rdma.wait_send()

            acc = jnp.sum(comm_ref[k].astype(jnp.float32), axis=0)
            if k < N_LAYERS - 1:
                xb = acc.astype(jnp.bfloat16)
            else:
                out_ref[...] = acc

    return pl.pallas_call(
        body,
        out_shape=jax.ShapeDtypeStruct((m, d), jnp.float32),
        in_specs=[pl.BlockSpec(memory_space=pltpu.VMEM)]
        + [pl.BlockSpec(memory_space=pl.ANY)] * 6,
        out_specs=pl.BlockSpec(memory_space=pltpu.VMEM),
        scratch_shapes=[
            pltpu.VMEM((N_LAYERS, d, f), jnp.float32),
            pltpu.VMEM((N_LAYERS, f, d), jnp.float32),
            pltpu.VMEM((N_LAYERS, N_DEV, m, d), jnp.bfloat16),
            pltpu.SemaphoreType.DMA((2 * N_LAYERS,)),
            pltpu.SemaphoreType.DMA((N_LAYERS, N_DEV - 1)),
            pltpu.SemaphoreType.DMA((N_LAYERS, N_DEV)),
        ],
        compiler_params=pltpu.CompilerParams(collective_id=0),
    )(x, Win0, Wout0, Win1, Wout1, Win2, Wout2)


# device time: 17958 ns/iter; 1.4763x vs baseline; 1.4763x over previous
import jax
import jax.numpy as jnp
from jax import lax
from jax.experimental import pallas as pl
from jax.experimental.pallas import tpu as pltpu

N_DEV = 4
N_LAYERS = 3


def kernel(x, Win0, Wout0, Win1, Wout1, Win2, Wout2):
    m, d = x.shape
    _, f = Win0.shape

    def body(x_ref, win0_ref, wout0_ref, win1_ref, wout1_ref, win2_ref,
             wout2_ref, out_ref, win_vmem, wout_vmem, comm_ref,
             load_sems, send_sems, recv_sems):
        my_pos = lax.axis_index("i")

        win_hbm = [win0_ref, win1_ref, win2_ref]
        wout_hbm = [wout0_ref, wout1_ref, wout2_ref]
        loads = []
        for k in range(N_LAYERS):
            ci = pltpu.make_async_copy(
                win_hbm[k], win_vmem.at[k], load_sems.at[2 * k])
            ci.start()
            co = pltpu.make_async_copy(
                wout_hbm[k], wout_vmem.at[k], load_sems.at[2 * k + 1])
            co.start()
            loads.append((ci, co))

        barrier_sem = pltpu.get_barrier_semaphore()
        for off in range(1, N_DEV):
            peer = (my_pos + off) % N_DEV
            pl.semaphore_signal(
                barrier_sem, inc=1,
                device_id=(peer,), device_id_type=pl.DeviceIdType.MESH,
            )
        pl.semaphore_wait(barrier_sem, N_DEV - 1)

        xb = x_ref[...].astype(jnp.bfloat16)
        loads[0][0].wait()
        wi = win_vmem[0].astype(jnp.bfloat16)
        loads[0][1].wait()
        wo = wout_vmem[0].astype(jnp.bfloat16)

        for k in range(N_LAYERS):
            h = jnp.dot(xb, wi, preferred_element_type=jnp.float32)
            h = jnp.maximum(h, 0.0).astype(jnp.bfloat16)
            p = jnp.dot(h, wo, preferred_element_type=jnp.float32)

            comm_ref[k, my_pos] = p.astype(jnp.bfloat16)
            sends = []
            for off in range(1, N_DEV):
                peer = (my_pos + off) % N_DEV
                rdma = pltpu.make_async_remote_copy(
                    src_ref=comm_ref.at[k, my_pos],
                    dst_ref=comm_ref.at[k, my_pos],
                    send_sem=send_sems.at[k, off - 1],
                    recv_sem=recv_sems.at[k, my_pos],
                    device_id=(peer,),
                    device_id_type=pl.DeviceIdType.MESH,
                )
                rdma.start()
                sends.append(rdma)

            if k + 1 < N_LAYERS:
                loads[k + 1][0].wait()
                wi = win_vmem[k + 1].astype(jnp.bfloat16)
                loads[k + 1][1].wait()
                wo = wout_vmem[k + 1].astype(jnp.bfloat16)

            for off in range(1, N_DEV):
                sender = (my_pos + off) % N_DEV
                recv = pltpu.make_async_remote_copy(
                    src_ref=comm_ref.at[k, sender],
                    dst_ref=comm_ref.at[k, sender],
                    send_sem=send_sems.at[k, off - 1],
                    recv_sem=recv_sems.at[k, sender],
                    device_id=(my_pos,),
                    device_id_type=pl.DeviceIdType.MESH,
                )
                recv.wait_recv()
            for rdma in sends:
                rdma.wait_send()

            acc = jnp.sum(comm_ref[k].astype(jnp.float32), axis=0)
            if k < N_LAYERS - 1:
                xb = acc.astype(jnp.bfloat16)
            else:
                out_ref[...] = acc

    hbm = lambda w: pltpu.with_memory_space_constraint(w, pltpu.MemorySpace.HBM)
    Win0, Wout0 = hbm(Win0), hbm(Wout0)
    Win1, Wout1 = hbm(Win1), hbm(Wout1)
    Win2, Wout2 = hbm(Win2), hbm(Wout2)

    return pl.pallas_call(
        body,
        out_shape=jax.ShapeDtypeStruct((m, d), jnp.float32),
        in_specs=[pl.BlockSpec(memory_space=pltpu.VMEM)]
        + [pl.BlockSpec(memory_space=pltpu.MemorySpace.HBM)] * 6,
        out_specs=pl.BlockSpec(memory_space=pltpu.VMEM),
        scratch_shapes=[
            pltpu.VMEM((N_LAYERS, d, f), jnp.float32),
            pltpu.VMEM((N_LAYERS, f, d), jnp.float32),
            pltpu.VMEM((N_LAYERS, N_DEV, m, d), jnp.bfloat16),
            pltpu.SemaphoreType.DMA((2 * N_LAYERS,)),
            pltpu.SemaphoreType.DMA((N_LAYERS, N_DEV - 1)),
            pltpu.SemaphoreType.DMA((N_LAYERS, N_DEV)),
        ],
        compiler_params=pltpu.CompilerParams(collective_id=0),
    )(x, Win0, Wout0, Win1, Wout1, Win2, Wout2)
